# Initial kernel scaffold; baseline (speedup 1.0000x reference)
import jax
import jax.numpy as jnp
from jax.experimental import pallas as pl


def kernel(x, tables):
    raise NotImplementedError("write your pallas kernel here")


# speedup vs baseline: 2.2684x; 2.2684x over previous
import functools

import numpy as np
import jax
import jax.numpy as jnp
from jax import lax
from jax.experimental import pallas as pl
from jax.experimental.pallas import tpu as pltpu
from jax.experimental.pallas import tpu_sc as plsc

_FEATURE_DIMS = [3847] * 26
_NUM_FIELDS = 26
_EMBED_DIM = 16
_TOTAL_ROWS = sum(_FEATURE_DIMS) + 1
_BATCH = 4096
_NPAIR = 325
_HALF = 328
_NIDX = 2 * _HALF
_QCHUNKS = _NIDX // 16

_NWORKERS = 32
_ROWS_PER_W = _BATCH // _NWORKERS

_i_idx, _j_idx = np.triu_indices(_NUM_FIELDS, k=1)


def _pad(v, n, fill=0):
    out = np.full((n,), fill, np.int32)
    out[: v.shape[0]] = v.astype(np.int32)
    return out


_COL = np.concatenate([_pad(_i_idx, _HALF), _pad(_j_idx, _HALF)])
_TBASE = np.concatenate([_pad(_j_idx * _TOTAL_ROWS, _HALF),
                         _pad(_i_idx * _TOTAL_ROWS, _HALF)])
_OFFSETS = np.concatenate([[0], np.cumsum(_FEATURE_DIMS)[:-1]]).astype(np.int32)

_CHUNKS = ((0, 128), (128, 128), (256, 72), (328, 128), (456, 128), (584, 72))


def _body(xa_hbm, col_hbm, tb_hbm, tab_hbm, out_hbm,
          xa_v, col_v, tb_v, idx_v, ga_v, gb_v, out_v, sem):
    cid = lax.axis_index("c")
    sid = lax.axis_index("s")
    wid = sid * 2 + cid
    base = wid * _ROWS_PER_W

    pltpu.sync_copy(
        xa_hbm.at[pl.ds(base * _NUM_FIELDS, _ROWS_PER_W * _NUM_FIELDS)], xa_v)
    pltpu.sync_copy(col_hbm, col_v)
    pltpu.sync_copy(tb_hbm, tb_v)

    def row_body(r, carry):
        def idx_body(q, c2):
            cols = col_v[pl.ds(q * 16, 16)]
            flat = cols + r * _NUM_FIELDS
            xvals = plsc.load_gather(xa_v, [flat])
            idx_v[pl.ds(q * 16, 16)] = xvals + tb_v[pl.ds(q * 16, 16)]
            return c2

        lax.fori_loop(0, _QCHUNKS, idx_body, 0)

        copies = []
        for off, cnt in _CHUNKS[:3]:
            copies.append(pltpu.async_copy(
                tab_hbm.at[idx_v.at[pl.ds(off, cnt)]],
                ga_v.at[pl.ds(off, cnt)], sem))
        for off, cnt in _CHUNKS[3:]:
            copies.append(pltpu.async_copy(
                tab_hbm.at[idx_v.at[pl.ds(off, cnt)]],
                gb_v.at[pl.ds(off - _HALF, cnt)], sem))
        for cpy in copies:
            cpy.wait()

        def mul_body(p, c2):
            out_v[pl.ds(p * 16, 16)] = ga_v[p] * gb_v[p]
            return c2

        lax.fori_loop(0, _NPAIR, mul_body, 0)

        pltpu.sync_copy(out_v, out_hbm.at[base + r])
        return carry

    lax.fori_loop(0, _ROWS_PER_W, row_body, 0)


@functools.cache
def _build_call():
    mesh = plsc.VectorSubcoreMesh(core_axis_name="c", subcore_axis_name="s")
    return pl.kernel(
        _body,
        mesh=mesh,
        compiler_params=pltpu.CompilerParams(
            needs_layout_passes=False, use_tc_tiling_on_sc=False),
        out_type=jax.ShapeDtypeStruct((_BATCH, _NPAIR * _EMBED_DIM),
                                      jnp.float32),
        scratch_types=[
            pltpu.VMEM((_ROWS_PER_W * _NUM_FIELDS,), jnp.int32),
            pltpu.VMEM((_NIDX,), jnp.int32),
            pltpu.VMEM((_NIDX,), jnp.int32),
            pltpu.VMEM((_NIDX,), jnp.int32),
            pltpu.VMEM((_HALF, _EMBED_DIM), jnp.float32),
            pltpu.VMEM((_HALF, _EMBED_DIM), jnp.float32),
            pltpu.VMEM((_NPAIR * _EMBED_DIM,), jnp.float32),
            pltpu.SemaphoreType.DMA,
        ],
    )


@jax.jit
def kernel(x, tables):
    xa = (x + jnp.asarray(_OFFSETS)[None, :]).reshape(-1)
    tab = tables.reshape(_NUM_FIELDS * _TOTAL_ROWS, _EMBED_DIM)
    out = _build_call()(xa, jnp.asarray(_COL), jnp.asarray(_TBASE), tab)
    return out.reshape(_BATCH, _NPAIR, _EMBED_DIM)


# speedup vs baseline: 2.3629x; 1.0417x over previous
import functools

import numpy as np
import jax
import jax.numpy as jnp
from jax import lax
from jax.experimental import pallas as pl
from jax.experimental.pallas import tpu as pltpu
from jax.experimental.pallas import tpu_sc as plsc

_FEATURE_DIMS = [3847] * 26
_NUM_FIELDS = 26
_EMBED_DIM = 16
_TOTAL_ROWS = sum(_FEATURE_DIMS) + 1
_BATCH = 4096
_NPAIR = 325
_HALF = 328
_NIDX = 2 * _HALF
_QCHUNKS = _NIDX // 16

_NWORKERS = 32
_ROWS_PER_W = _BATCH // _NWORKERS
_NBUF = 4
_OUT_PAD = _HALF * _EMBED_DIM
_OUT_LEN = _NPAIR * _EMBED_DIM

_i_idx, _j_idx = np.triu_indices(_NUM_FIELDS, k=1)


def _pad(v, n, fill=0):
    out = np.full((n,), fill, np.int32)
    out[: v.shape[0]] = v.astype(np.int32)
    return out


_COL = np.concatenate([_pad(_i_idx, _HALF), _pad(_j_idx, _HALF)])
_TBASE = np.concatenate([_pad(_j_idx * _TOTAL_ROWS, _HALF),
                         _pad(_i_idx * _TOTAL_ROWS, _HALF)])
_OFFSETS = np.concatenate([[0], np.cumsum(_FEATURE_DIMS)[:-1]]).astype(np.int32)

_CHUNKS = ((0, 128), (128, 128), (256, 72), (328, 128), (456, 128), (584, 72))


def _body(xa_hbm, col_hbm, tb_hbm, tab_hbm, out_hbm, *refs):
    xa_v, col_v, tb_v = refs[0:3]
    idx_b = refs[3:3 + _NBUF]
    ga_b = refs[3 + _NBUF:3 + 2 * _NBUF]
    gb_b = refs[3 + 2 * _NBUF:3 + 3 * _NBUF]
    out_b = refs[3 + 3 * _NBUF:5 + 3 * _NBUF]
    gsem = refs[5 + 3 * _NBUF:5 + 4 * _NBUF]
    osem = refs[5 + 4 * _NBUF:7 + 4 * _NBUF]

    cid = lax.axis_index("c")
    sid = lax.axis_index("s")
    wid = sid * 2 + cid
    base = wid * _ROWS_PER_W

    pltpu.sync_copy(
        xa_hbm.at[pl.ds(base * _NUM_FIELDS, _ROWS_PER_W * _NUM_FIELDS)], xa_v)
    pltpu.sync_copy(col_hbm, col_v)
    pltpu.sync_copy(tb_hbm, tb_v)

    def fire_row(r, b):
        idx_v, ga_v, gb_v = idx_b[b], ga_b[b], gb_b[b]

        def idx_body(q, c2):
            cols = col_v[pl.ds(q * 16, 16)]
            xvals = plsc.load_gather(xa_v, [cols + r * _NUM_FIELDS])
            idx_v[pl.ds(q * 16, 16)] = xvals + tb_v[pl.ds(q * 16, 16)]
            return c2

        lax.fori_loop(0, _QCHUNKS, idx_body, 0)
        for off, cnt in _CHUNKS[:3]:
            pltpu.async_copy(tab_hbm.at[idx_v.at[pl.ds(off, cnt)]],
                             ga_v.at[pl.ds(off, cnt)], gsem[b])
        for off, cnt in _CHUNKS[3:]:
            pltpu.async_copy(tab_hbm.at[idx_v.at[pl.ds(off, cnt)]],
                             gb_v.at[pl.ds(off - _HALF, cnt)], gsem[b])

    for b in range(_NBUF):
        fire_row(b, b)

    def outer(go, carry):
        g = go * _NBUF
        for b in range(_NBUF):
            r = g + b
            ga_v, gb_v, out_v = ga_b[b], gb_b[b], out_b[b % 2]
            pltpu.make_async_copy(
                tab_hbm.at[pl.ds(0, _HALF)], ga_v, gsem[b]).wait()
            pltpu.make_async_copy(
                tab_hbm.at[pl.ds(0, _HALF)], gb_v, gsem[b]).wait()

            @pl.when(r >= 2)
            def _():
                pltpu.make_async_copy(
                    out_v.at[pl.ds(0, _OUT_LEN)], out_hbm.at[base],
                    osem[b % 2]).wait()

            def mul_body(i, c2):
                for u in range(4):
                    p = i * 4 + u
                    out_v[pl.ds(p * 16, 16)] = ga_v[p] * gb_v[p]
                return c2

            lax.fori_loop(0, _HALF // 4, mul_body, 0)
            pltpu.async_copy(out_v.at[pl.ds(0, _OUT_LEN)],
                             out_hbm.at[base + r], osem[b % 2])

            @pl.when(r + _NBUF < _ROWS_PER_W)
            def _():
                fire_row(r + _NBUF, b)
        return carry

    lax.fori_loop(0, _ROWS_PER_W // _NBUF, outer, 0)

    for b in range(2):
        pltpu.make_async_copy(out_b[b].at[pl.ds(0, _OUT_LEN)],
                              out_hbm.at[base], osem[b]).wait()


@functools.cache
def _build_call():
    mesh = plsc.VectorSubcoreMesh(core_axis_name="c", subcore_axis_name="s")
    scratch = [
        pltpu.VMEM((_ROWS_PER_W * _NUM_FIELDS,), jnp.int32),
        pltpu.VMEM((_NIDX,), jnp.int32),
        pltpu.VMEM((_NIDX,), jnp.int32),
    ]
    scratch += [pltpu.VMEM((_NIDX,), jnp.int32) for _ in range(_NBUF)]
    scratch += [pltpu.VMEM((_HALF, _EMBED_DIM), jnp.float32)
                for _ in range(2 * _NBUF)]
    scratch += [pltpu.VMEM((_OUT_PAD,), jnp.float32) for _ in range(2)]
    scratch += [pltpu.SemaphoreType.DMA for _ in range(_NBUF + 2)]
    return pl.kernel(
        _body,
        mesh=mesh,
        compiler_params=pltpu.CompilerParams(
            needs_layout_passes=False, use_tc_tiling_on_sc=False),
        out_type=jax.ShapeDtypeStruct((_BATCH, _OUT_LEN), jnp.float32),
        scratch_types=scratch,
    )


@jax.jit
def kernel(x, tables):
    xa = (x + jnp.asarray(_OFFSETS)[None, :]).reshape(-1)
    tab = tables.reshape(_NUM_FIELDS * _TOTAL_ROWS, _EMBED_DIM)
    out = _build_call()(xa, jnp.asarray(_COL), jnp.asarray(_TBASE), tab)
    return out.reshape(_BATCH, _NPAIR, _EMBED_DIM)
